# baseline (speedup 1.0000x reference)
import functools

import jax
import jax.numpy as jnp
from jax.experimental import pallas as pl
from jax.experimental.pallas import tpu as pltpu


def _fused_se_kernel(x_ref, w1_ref, b1_ref, w2_ref, b2_ref, o_ref, *,
                     inv_hw):
    H, W, C = x_ref.shape[1:]
    mid = w1_ref.shape[0]
    x = x_ref[0].reshape(H * W, C)
    pooled = (jnp.sum(x.astype(jnp.float32), axis=0) * inv_hw)[None, :]

    y1 = jax.lax.dot_general(pooled, w1_ref[...], (((1,), (1,)), ((), ())),
                             preferred_element_type=jnp.float32)
    y1 = jnp.maximum(y1 + b1_ref[...].reshape(1, mid), 0.0)
    y2 = jax.lax.dot_general(y1, w2_ref[...], (((1,), (1,)), ((), ())),
                             preferred_element_type=jnp.float32)
    gate = jax.nn.sigmoid(y2 + b2_ref[...].reshape(1, C))
    gate = gate.astype(o_ref.dtype)

    o_ref[...] = x_ref[...] * gate[0][None, None, None, :]


@jax.jit
def _ca_fused(x, w1, b1, w2, b2):
    B, C, H, W = x.shape
    mid = w1.shape[0]
    xt = jnp.transpose(x, (0, 2, 3, 1))
    inv_hw = 1.0 / float(H * W)

    out = pl.pallas_call(
        functools.partial(_fused_se_kernel, inv_hw=inv_hw),
        out_shape=jax.ShapeDtypeStruct((B, H, W, C), x.dtype),
        grid=(B,),
        in_specs=[
            pl.BlockSpec((1, H, W, C), lambda b: (b, 0, 0, 0)),
            pl.BlockSpec((mid, C), lambda b: (0, 0)),
            pl.BlockSpec((mid, 1), lambda b: (0, 0)),
            pl.BlockSpec((C, mid), lambda b: (0, 0)),
            pl.BlockSpec((C, 1), lambda b: (0, 0)),
        ],
        out_specs=pl.BlockSpec((1, H, W, C), lambda b: (b, 0, 0, 0)),
        compiler_params=pltpu.CompilerParams(
            dimension_semantics=("parallel",)),
    )(xt, w1, b1, w2, b2)

    return jnp.transpose(out, (0, 3, 1, 2))


def kernel(x, w1, b1, w2, b2):
    return _ca_fused(x, w1, b1, w2, b2)
